# Initial kernel scaffold; baseline (speedup 1.0000x reference)
import jax
import jax.numpy as jnp
from jax.experimental import pallas as pl


def kernel(x, edge_index, embed_w, embed_b, lin_w, lin_b, att, layer_bias, ln_g, ln_b):
    raise NotImplementedError("write your pallas kernel here")


# speedup vs baseline: 17.0007x; 17.0007x over previous
import functools

import jax
import jax.numpy as jnp
from jax import lax
from jax.experimental import pallas as pl
from jax.experimental.pallas import tpu as pltpu
from jax.experimental.pallas import tpu_sc as plsc

N = 10000
E = 320000
D = 128
H = 8
C = 16
L = 5

NC = 2
NS = 16
NW = NC * NS
EW = E // NW
B = 80
NB = EW // B
WS = 256
WD = 128
WM = 128
ND = 800
CH = 400
NCH = N // CH

BN = 400
GN = N // BN
BE = 2000
GE = E // BE

_f32 = jnp.float32


def _embed_body(x_ref, wt_ref, b_ref, o_ref):
    h = jnp.dot(x_ref[...], wt_ref[...], preferred_element_type=_f32)
    o_ref[...] = jnp.maximum(h + b_ref[...], 0.0)


def _tc_embed(x, wt, b):
    return pl.pallas_call(
        _embed_body,
        grid=(GN,),
        in_specs=[
            pl.BlockSpec((BN, D), lambda i: (i, 0)),
            pl.BlockSpec((D, D), lambda i: (0, 0)),
            pl.BlockSpec((1, D), lambda i: (0, 0)),
        ],
        out_specs=pl.BlockSpec((BN, D), lambda i: (i, 0)),
        out_shape=jax.ShapeDtypeStruct((N, D), _f32),
    )(x, wt, b)


def _transform_body(h_ref, wt_ref, b_ref, a2_ref, st_ref, aa_ref):
    hs = jnp.dot(h_ref[...], wt_ref[...], preferred_element_type=_f32)
    hs = hs + b_ref[...]
    aa = jnp.dot(hs, a2_ref[...], preferred_element_type=_f32)
    st_ref[...] = jnp.concatenate(
        [hs, aa[:, H:2 * H], jnp.zeros((BN, WS - D - H), _f32)], axis=1)
    aa_ref[...] = aa


def _tc_transform(h, wt, b, a2):
    return pl.pallas_call(
        _transform_body,
        grid=(GN,),
        in_specs=[
            pl.BlockSpec((BN, D), lambda i: (i, 0)),
            pl.BlockSpec((D, D), lambda i: (0, 0)),
            pl.BlockSpec((1, D), lambda i: (0, 0)),
            pl.BlockSpec((D, 2 * H), lambda i: (0, 0)),
        ],
        out_specs=[
            pl.BlockSpec((BN, WS), lambda i: (i, 0)),
            pl.BlockSpec((BN, 2 * H), lambda i: (i, 0)),
        ],
        out_shape=[
            jax.ShapeDtypeStruct((N, WS), _f32),
            jax.ShapeDtypeStruct((N, 2 * H), _f32),
        ],
    )(h, wt, b, a2)


def _rowmax_body(aa_ref, dt_ref):
    aa = aa_ref[...]
    mj = jnp.max(aa[:, H:2 * H], axis=0, keepdims=True)
    mt = aa[:, 0:H] + mj
    mt = jnp.maximum(mt, 0.2 * mt)
    dt_ref[...] = jnp.concatenate(
        [aa[:, 0:H], mt, jnp.zeros((N, WD - 2 * H), _f32)], axis=1)


def _tc_rowmax(aa):
    return pl.pallas_call(
        _rowmax_body,
        out_shape=jax.ShapeDtypeStruct((N, WD), _f32),
    )(aa)


def _edge_body(xs_ref, xd_ref, dd_ref, r_ref, m_ref, du_ref):
    xs = xs_ref[...]
    xd = xd_ref[...]
    s = xd[:, 0:H] + xs[:, D:D + H]
    alpha = jnp.maximum(s, 0.2 * s)
    ex = jnp.exp(jnp.minimum(alpha - xd[:, H:2 * H], 0.0))
    gain = jnp.dot(ex, r_ref[...], preferred_element_type=_f32)
    m_ref[...] = xs[:, 0:D] * gain
    dd = dd_ref[0, 0, :]
    sl = lax.rem(dd, 16)
    oh = (sl[:, None] ==
          lax.broadcasted_iota(jnp.int32, (BE, 16), 1)).astype(_f32)
    du_ref[...] = (oh[:, :, None] * ex[:, None, :]).reshape(BE, WM)


def _tc_edge(xs, xd, dd, r):
    return pl.pallas_call(
        _edge_body,
        grid=(GE,),
        in_specs=[
            pl.BlockSpec((BE, WS), lambda i: (i, 0)),
            pl.BlockSpec((BE, WD), lambda i: (i, 0)),
            pl.BlockSpec((1, 1, BE), lambda i: (i, 0, 0)),
            pl.BlockSpec((H, D), lambda i: (0, 0)),
        ],
        out_specs=[
            pl.BlockSpec((BE, WM), lambda i: (i, 0)),
            pl.BlockSpec((BE, WM), lambda i: (i, 0)),
        ],
        out_shape=[
            jax.ShapeDtypeStruct((E, WM), _f32),
            jax.ShapeDtypeStruct((E, WM), _f32),
        ],
    )(xs, xd, dd, r)


def _combine_body(p0_ref, p1_ref, d0_ref, d1_ref, bias_ref, g_ref, bb_ref,
                  r_ref, o_ref):
    msg = p0_ref[...] + p1_ref[...]
    den = d0_ref[...] + d1_ref[...]
    inv = 1.0 / (den + 1e-16)
    drep = jnp.dot(inv, r_ref[...], preferred_element_type=_f32)
    y = msg * drep + bias_ref[...]
    y = jnp.where(y > 0.0, y, jnp.exp(jnp.minimum(y, 0.0)) - 1.0)
    mu = jnp.mean(y, axis=-1, keepdims=True)
    var = jnp.mean((y - mu) ** 2, axis=-1, keepdims=True)
    o_ref[...] = (y - mu) / jnp.sqrt(var + 1e-5) * g_ref[...] + bb_ref[...]


def _tc_combine(p0, p1, d0, d1, bias, g, bb, r):
    return pl.pallas_call(
        _combine_body,
        grid=(GN,),
        in_specs=[
            pl.BlockSpec((BN, WM), lambda i: (i, 0)),
            pl.BlockSpec((BN, WM), lambda i: (i, 0)),
            pl.BlockSpec((BN, H), lambda i: (i, 0)),
            pl.BlockSpec((BN, H), lambda i: (i, 0)),
            pl.BlockSpec((1, D), lambda i: (0, 0)),
            pl.BlockSpec((1, D), lambda i: (0, 0)),
            pl.BlockSpec((1, D), lambda i: (0, 0)),
            pl.BlockSpec((H, D), lambda i: (0, 0)),
        ],
        out_specs=pl.BlockSpec((BN, D), lambda i: (i, 0)),
        out_shape=jax.ShapeDtypeStruct((N, D), _f32),
    )(p0, p1, d0, d1, bias, g, bb, r)


@functools.cache
def _sc_mesh():
    return plsc.VectorSubcoreMesh(core_axis_name="c", subcore_axis_name="s",
                                  num_cores=NC, num_subcores=NS)


def _sc_gather_body(st_hbm, dt_hbm, sidx_hbm, didx_hbm, xs_hbm, xd_hbm,
                    sidx_v, didx_v, sbuf, dbuf, sem1, sem2):
    c = lax.axis_index("c")
    s = lax.axis_index("s")
    w = s * NC + c
    pltpu.sync_copy(sidx_hbm.at[w], sidx_v)
    pltpu.sync_copy(didx_hbm.at[w], didx_v)

    def body(j, carry):
        cp1 = pltpu.async_copy(st_hbm.at[sidx_v.at[j]], sbuf, sem1)
        cp2 = pltpu.async_copy(dt_hbm.at[didx_v.at[j]], dbuf, sem2)
        cp1.wait()
        cp2.wait()
        pltpu.sync_copy(sbuf, xs_hbm.at[w * NB + j])
        pltpu.sync_copy(dbuf, xd_hbm.at[w * NB + j])
        return carry

    lax.fori_loop(0, NB, body, 0)


@functools.cache
def _sc_gather():
    return pl.kernel(
        _sc_gather_body,
        out_type=(
            jax.ShapeDtypeStruct((NW * NB, B, WS), _f32),
            jax.ShapeDtypeStruct((NW * NB, B, WD), _f32),
        ),
        mesh=_sc_mesh(),
        scratch_types=(
            pltpu.VMEM((NB, B), jnp.int32),
            pltpu.VMEM((NB, B), jnp.int32),
            pltpu.VMEM((B, WS), _f32),
            pltpu.VMEM((B, WD), _f32),
            pltpu.SemaphoreType.DMA,
            pltpu.SemaphoreType.DMA,
        ),
    )


def _sc_scatter_body(msg_hbm, den_hbm, didx_hbm, d16_hbm, outm_hbm, outd_hbm,
                     didx_v, d16_v, mbuf, accm, accd):
    c = lax.axis_index("c")
    s = lax.axis_index("s")
    w = s * NC + c

    pltpu.sync_copy(didx_hbm.at[w], didx_v)
    pltpu.sync_copy(d16_hbm.at[w], d16_v)

    def zrow(rr, carry):
        for k in range(WM // 16):
            mbuf[rr, pl.ds(k * 16, 16)] = jnp.zeros((16,), _f32)
        return carry

    lax.fori_loop(0, B, zrow, 0)

    for i in range(9):
        k = s + NS * i

        @pl.when(k < N // B)
        def _():
            off = pl.multiple_of(k * B, 8)
            pltpu.sync_copy(mbuf, accm.at[pl.ds(off, B)])

        @pl.when(jnp.logical_and(k >= N // B, k < N // B + ND // B))
        def _():
            off = pl.multiple_of((k - N // B) * B, 8)
            pltpu.sync_copy(mbuf, accd.at[pl.ds(off, B)])

    plsc.subcore_barrier()

    def body(j, carry):
        pltpu.sync_copy(msg_hbm.at[w * NB + j], mbuf)
        pltpu.sync_copy(mbuf, accm.at[didx_v.at[j]], add=True)
        pltpu.sync_copy(den_hbm.at[w * NB + j], mbuf)
        pltpu.sync_copy(mbuf, accd.at[d16_v.at[j]], add=True)
        return carry

    lax.fori_loop(0, NB, body, 0)
    plsc.subcore_barrier()
    for i in range(9):
        k = s + NS * i

        @pl.when(k < N // B)
        def _():
            off = pl.multiple_of(k * B, 8)
            pltpu.sync_copy(accm.at[pl.ds(off, B)], mbuf)
            pltpu.sync_copy(mbuf, outm_hbm.at[c, pl.ds(off, B)])

        @pl.when(jnp.logical_and(k >= N // B, k < N // B + ND // B))
        def _():
            off = pl.multiple_of((k - N // B) * B, 8)
            pltpu.sync_copy(accd.at[pl.ds(off, B)], mbuf)
            pltpu.sync_copy(mbuf, outd_hbm.at[c, pl.ds(off, B)])


@functools.cache
def _sc_scatter():
    return pl.kernel(
        _sc_scatter_body,
        out_type=(
            jax.ShapeDtypeStruct((NC, N, WM), _f32),
            jax.ShapeDtypeStruct((NC, ND, WM), _f32),
        ),
        mesh=_sc_mesh(),
        scratch_types=(
            pltpu.VMEM((NB, B), jnp.int32),
            pltpu.VMEM((NB, B), jnp.int32),
            pltpu.VMEM((B, WM), _f32),
            pltpu.VMEM_SHARED((N, WM), _f32),
            pltpu.VMEM_SHARED((ND, WM), _f32),
        ),
    )


def kernel(x, edge_index, embed_w, embed_b, lin_w, lin_b, att, layer_bias,
           ln_g, ln_b):
    src = edge_index[0].reshape(NW, NB, B)
    dst = edge_index[1].reshape(NW, NB, B)
    dst16 = edge_index[1] // 16
    dst16 = dst16.reshape(NW, NB, B)
    dst_tc = edge_index[1].reshape(GE, 1, BE)

    eye = jnp.eye(H, dtype=_f32)
    a_dst = (att[:, :, :C, None] * eye[None, :, None, :]).reshape(L, D, H)
    a_src = (att[:, :, C:, None] * eye[None, :, None, :]).reshape(L, D, H)
    a2 = jnp.concatenate([a_dst, a_src], axis=-1)

    r = jnp.repeat(eye, C, axis=1)

    h = _tc_embed(x, embed_w.T, embed_b.reshape(1, D))
    for l in range(L):
        st, aa = _tc_transform(h, lin_w[l].T, lin_b[l].reshape(1, D), a2[l])
        dt = _tc_rowmax(aa)
        xs, xd = _sc_gather()(st, dt, src, dst)
        msg, du = _tc_edge(xs.reshape(E, WS), xd.reshape(E, WD), dst_tc, r)
        pm, pd = _sc_scatter()(msg.reshape(NW * NB, B, WM),
                               du.reshape(NW * NB, B, WM), dst, dst16)
        den = pd[:, :N // 16, :].reshape(NC, N, H)
        h = _tc_combine(pm[0], pm[1], den[0], den[1],
                        layer_bias[l].reshape(1, D),
                        ln_g[l].reshape(1, D), ln_b[l].reshape(1, D), r)
    return h


# speedup vs baseline: 45.3414x; 2.6670x over previous
import functools

import jax
import jax.numpy as jnp
from jax import lax
from jax.experimental import pallas as pl
from jax.experimental.pallas import tpu as pltpu
from jax.experimental.pallas import tpu_sc as plsc

N = 10000
E = 320000
D = 128
H = 8
C = 16
L = 5

NC = 2
NS = 16
NW = NC * NS
EW = E // NW
NBK = EW // 16
GB = 16
WS = 256
WD = 128
WM = 128
ND = 640
TCHUNK = (N + ND) // GB // NS + 1

BN = 400
GN = N // BN

_f32 = jnp.float32


def _embed_body(x_ref, wt_ref, b_ref, o_ref):
    h = jnp.dot(x_ref[...], wt_ref[...], preferred_element_type=_f32)
    o_ref[...] = jnp.maximum(h + b_ref[...], 0.0)


def _tc_embed(x, wt, b):
    return pl.pallas_call(
        _embed_body,
        grid=(GN,),
        in_specs=[
            pl.BlockSpec((BN, D), lambda i: (i, 0)),
            pl.BlockSpec((D, D), lambda i: (0, 0)),
            pl.BlockSpec((1, D), lambda i: (0, 0)),
        ],
        out_specs=pl.BlockSpec((BN, D), lambda i: (i, 0)),
        out_shape=jax.ShapeDtypeStruct((N, D), _f32),
    )(x, wt, b)


def _transform_body(h_ref, wt_ref, b_ref, a2_ref, st_ref, aa_ref):
    hs = jnp.dot(h_ref[...], wt_ref[...], preferred_element_type=_f32)
    hs = hs + b_ref[...]
    aa = jnp.dot(hs, a2_ref[...], preferred_element_type=_f32)
    st_ref[...] = jnp.concatenate(
        [hs, aa[:, H:2 * H], jnp.zeros((BN, WS - D - H), _f32)], axis=1)
    aa_ref[...] = aa


def _tc_transform(h, wt, b, a2):
    return pl.pallas_call(
        _transform_body,
        grid=(GN,),
        in_specs=[
            pl.BlockSpec((BN, D), lambda i: (i, 0)),
            pl.BlockSpec((D, D), lambda i: (0, 0)),
            pl.BlockSpec((1, D), lambda i: (0, 0)),
            pl.BlockSpec((D, 2 * H), lambda i: (0, 0)),
        ],
        out_specs=[
            pl.BlockSpec((BN, WS), lambda i: (i, 0)),
            pl.BlockSpec((BN, 2 * H), lambda i: (i, 0)),
        ],
        out_shape=[
            jax.ShapeDtypeStruct((N, WS), _f32),
            jax.ShapeDtypeStruct((N, 2 * H), _f32),
        ],
    )(h, wt, b, a2)


def _rowmax_body(aa_ref, dt_ref):
    aa = aa_ref[...]
    mj = jnp.max(aa[:, H:2 * H], axis=0, keepdims=True)
    mt = aa[:, 0:H] + mj
    mt = jnp.maximum(mt, 0.2 * mt)
    dt_ref[...] = jnp.concatenate(
        [aa[:, 0:H], mt, jnp.zeros((N, WD - 2 * H), _f32)], axis=1)


def _tc_rowmax(aa):
    return pl.pallas_call(
        _rowmax_body,
        out_shape=jax.ShapeDtypeStruct((N, WD), _f32),
    )(aa)


def _combine_body(p0_ref, p1_ref, d0_ref, d1_ref, bias_ref, g_ref, bb_ref,
                  r_ref, o_ref):
    msg = p0_ref[...] + p1_ref[...]
    den = d0_ref[...] + d1_ref[...]
    inv = 1.0 / (den + 1e-16)
    drep = jnp.dot(inv, r_ref[...], preferred_element_type=_f32)
    y = msg * drep + bias_ref[...]
    y = jnp.where(y > 0.0, y, jnp.exp(jnp.minimum(y, 0.0)) - 1.0)
    mu = jnp.mean(y, axis=-1, keepdims=True)
    var = jnp.mean((y - mu) ** 2, axis=-1, keepdims=True)
    o_ref[...] = (y - mu) / jnp.sqrt(var + 1e-5) * g_ref[...] + bb_ref[...]


def _tc_combine(p0, p1, d0, d1, bias, g, bb, r):
    return pl.pallas_call(
        _combine_body,
        grid=(GN,),
        in_specs=[
            pl.BlockSpec((BN, WM), lambda i: (i, 0)),
            pl.BlockSpec((BN, WM), lambda i: (i, 0)),
            pl.BlockSpec((BN, H), lambda i: (i, 0)),
            pl.BlockSpec((BN, H), lambda i: (i, 0)),
            pl.BlockSpec((1, D), lambda i: (0, 0)),
            pl.BlockSpec((1, D), lambda i: (0, 0)),
            pl.BlockSpec((1, D), lambda i: (0, 0)),
            pl.BlockSpec((H, D), lambda i: (0, 0)),
        ],
        out_specs=pl.BlockSpec((BN, D), lambda i: (i, 0)),
        out_shape=jax.ShapeDtypeStruct((N, D), _f32),
    )(p0, p1, d0, d1, bias, g, bb, r)


@functools.cache
def _sc_mesh():
    return plsc.VectorSubcoreMesh(core_axis_name="c", subcore_axis_name="s",
                                  num_cores=NC, num_subcores=NS)


def _sc_edge_body(st_hbm, dt_hbm, sidx_hbm, didx_hbm, outm_hbm, outd_hbm,
                  sidx_v, didx_v, dring, d16ring, sbuf, dbuf, mbufv, denbuf,
                  accm, accd, *sems):
    c = lax.axis_index("c")
    s = lax.axis_index("s")
    w = s * NC + c
    sem_sg = sems[0:3]
    sem_dg = sems[3:6]

    pltpu.sync_copy(sidx_hbm.at[w], sidx_v)
    pltpu.sync_copy(didx_hbm.at[w], didx_v)

    def zrow(rr, carry):
        for k in range(WM // 16):
            dbuf[0, rr, pl.ds(k * 16, 16)] = jnp.zeros((16,), _f32)
        return carry

    lax.fori_loop(0, GB, zrow, 0)

    for i in range(TCHUNK):
        k = s + NS * i

        @pl.when(k < N // GB)
        def _():
            off = pl.multiple_of(k * GB, 8)
            pltpu.sync_copy(dbuf.at[0], accm.at[pl.ds(off, GB)])

        @pl.when(jnp.logical_and(k >= N // GB, k < (N + ND) // GB))
        def _():
            off = pl.multiple_of((k - N // GB) * GB, 8)
            pltpu.sync_copy(dbuf.at[0], accd.at[pl.ds(off, GB)])

    plsc.subcore_barrier()

    def gissue(j, buf):
        sv = sidx_v.at[pl.ds(j * 16, 16)]
        dv = didx_v.at[pl.ds(j * 16, 16)]
        pltpu.async_copy(st_hbm.at[sv], sbuf.at[buf], sem_sg[buf])
        pltpu.async_copy(dt_hbm.at[dv], dbuf.at[buf], sem_dg[buf])

    def gwait(j, buf):
        sv = sidx_v.at[pl.ds(j * 16, 16)]
        dv = didx_v.at[pl.ds(j * 16, 16)]
        pltpu.make_async_copy(st_hbm.at[sv], sbuf.at[buf],
                              sem_sg[buf]).wait()
        pltpu.make_async_copy(dt_hbm.at[dv], dbuf.at[buf],
                              sem_dg[buf]).wait()

    gissue(0, 0)
    gissue(1, 1)

    def one(j, cur):
        gwait(j, cur)
        dv = didx_v[pl.ds(j * 16, 16)]
        dring[cur, :] = dv
        d16ring[cur, :] = dv >> 4
        iota = lax.iota(jnp.int32, 16)
        idx_m = (iota & 7) + 8
        idx_s = iota & 7
        lof = jnp.minimum(jnp.maximum(8 - iota, 0), 1).astype(_f32)
        hif = 1.0 - lof

        def edge(k, carry):
            dvec = dbuf[cur, k, 0:16]
            ajv = sbuf[cur, k, pl.ds(D, 16)]
            s16 = dvec + ajv
            alpha = jnp.maximum(s16, 0.2 * s16)
            mperm = jnp.take_along_axis(s16, idx_m, axis=0)
            ex16 = jnp.exp(jnp.minimum(alpha - mperm, 0.0))
            nv = jnp.take_along_axis(dv, jnp.full((16,), k, jnp.int32),
                                     axis=0)
            slv = nv & 15
            exlo = ex16 * lof
            exhi = jnp.take_along_axis(ex16, idx_s, axis=0) * hif
            pv = (slv & 1).astype(_f32)
            val = exhi * pv + exlo * (1.0 - pv)
            qv = slv >> 1
            for q in range(8):
                qd = qv - q
                mq = (1 - jnp.minimum(qd * qd, 1)).astype(_f32)
                denbuf[k, pl.ds(16 * q, 16)] = val * mq
            for hh in range(8):
                g = jnp.take_along_axis(
                    ex16, jnp.full((16,), hh, jnp.int32), axis=0)
                seg = sbuf[cur, k, pl.ds(16 * hh, 16)]
                mbufv[k, pl.ds(16 * hh, 16)] = seg * g
            return carry

        lax.fori_loop(0, 16, edge, 0)

        pltpu.sync_copy(mbufv, accm.at[dring.at[cur]], add=True)
        pltpu.sync_copy(denbuf, accd.at[d16ring.at[cur]], add=True)

        prev = (cur + 2) % 3

        @pl.when(j + 2 < NBK)
        def _():
            gissue(j + 2, prev)

    def body(jj, carry):
        for t in range(3):
            one(jj * 3 + t, t)
        return carry

    lax.fori_loop(0, NBK // 3, body, 0)
    for jt in range(NBK - 3 * (NBK // 3)):
        one(3 * (NBK // 3) + jt, jt)
    plsc.subcore_barrier()

    for i in range(TCHUNK):
        k = s + NS * i

        @pl.when(k < N // GB)
        def _():
            off = pl.multiple_of(k * GB, 8)
            pltpu.sync_copy(accm.at[pl.ds(off, GB)], dbuf.at[0])
            pltpu.sync_copy(dbuf.at[0], outm_hbm.at[c, pl.ds(off, GB)])

        @pl.when(jnp.logical_and(k >= N // GB, k < (N + ND) // GB))
        def _():
            off = pl.multiple_of((k - N // GB) * GB, 8)
            pltpu.sync_copy(accd.at[pl.ds(off, GB)], dbuf.at[0])
            pltpu.sync_copy(dbuf.at[0], outd_hbm.at[c, pl.ds(off, GB)])


@functools.cache
def _sc_edge():
    return pl.kernel(
        _sc_edge_body,
        out_type=(
            jax.ShapeDtypeStruct((NC, N, WM), _f32),
            jax.ShapeDtypeStruct((NC, ND, WM), _f32),
        ),
        mesh=_sc_mesh(),
        scratch_types=(
            pltpu.VMEM((EW,), jnp.int32),
            pltpu.VMEM((EW,), jnp.int32),
            pltpu.VMEM((3, 16), jnp.int32),
            pltpu.VMEM((3, 16), jnp.int32),
            pltpu.VMEM((3, 16, WS), _f32),
            pltpu.VMEM((3, 16, WM), _f32),
            pltpu.VMEM((16, WM), _f32),
            pltpu.VMEM((16, WM), _f32),
            pltpu.VMEM_SHARED((N, WM), _f32),
            pltpu.VMEM_SHARED((ND, WM), _f32),
        ) + (pltpu.SemaphoreType.DMA,) * 6,
    )


def kernel(x, edge_index, embed_w, embed_b, lin_w, lin_b, att, layer_bias,
           ln_g, ln_b):
    src16 = edge_index[0].reshape(NW, EW)
    dst16g = edge_index[1].reshape(NW, EW)

    eye = jnp.eye(H, dtype=_f32)
    a_dst = (att[:, :, :C, None] * eye[None, :, None, :]).reshape(L, D, H)
    a_src = (att[:, :, C:, None] * eye[None, :, None, :]).reshape(L, D, H)
    a2 = jnp.concatenate([a_dst, a_src], axis=-1)

    r = jnp.repeat(eye, C, axis=1)

    h = _tc_embed(x, embed_w.T, embed_b.reshape(1, D))
    for l in range(L):
        st, aa = _tc_transform(h, lin_w[l].T, lin_b[l].reshape(1, D), a2[l])
        dt = _tc_rowmax(aa)
        pm, pd = _sc_edge()(st, dt, src16, dst16g)
        den = pd[:, :N // 16, :].reshape(NC, N, H)
        h = _tc_combine(pm[0], pm[1], den[0], den[1],
                        layer_bias[l].reshape(1, D),
                        ln_g[l].reshape(1, D), ln_b[l].reshape(1, D), r)
    return h


# speedup vs baseline: 55.4322x; 1.2226x over previous
import functools

import jax
import jax.numpy as jnp
from jax import lax
from jax.experimental import pallas as pl
from jax.experimental.pallas import tpu as pltpu
from jax.experimental.pallas import tpu_sc as plsc

N = 10000
E = 320000
D = 128
H = 8
C = 16
L = 5

NC = 2
NS = 16
NW = NC * NS
EW = E // NW
NBK = EW // 16
GB = 16
WS = 256
WD = 128
WM = 128
ND = 640
TCHUNK = (N + ND) // GB // NS + 1

BN = 400
GN = N // BN

_f32 = jnp.float32


def _embed_body(x_ref, wt_ref, b_ref, o_ref):
    h = jnp.dot(x_ref[...], wt_ref[...], preferred_element_type=_f32)
    o_ref[...] = jnp.maximum(h + b_ref[...], 0.0)


def _tc_embed(x, wt, b):
    return pl.pallas_call(
        _embed_body,
        grid=(GN,),
        in_specs=[
            pl.BlockSpec((BN, D), lambda i: (i, 0)),
            pl.BlockSpec((D, D), lambda i: (0, 0)),
            pl.BlockSpec((1, D), lambda i: (0, 0)),
        ],
        out_specs=pl.BlockSpec((BN, D), lambda i: (i, 0)),
        out_shape=jax.ShapeDtypeStruct((N, D), _f32),
    )(x, wt, b)


def _transform_body(h_ref, wt_ref, b_ref, a2_ref, st_ref, aa_ref):
    hs = jnp.dot(h_ref[...], wt_ref[...], preferred_element_type=_f32)
    hs = hs + b_ref[...]
    aa = jnp.dot(hs, a2_ref[...], preferred_element_type=_f32)
    st_ref[...] = jnp.concatenate(
        [hs, aa[:, H:2 * H], jnp.zeros((BN, WS - D - H), _f32)], axis=1)
    aa_ref[...] = aa


def _tc_transform(h, wt, b, a2):
    return pl.pallas_call(
        _transform_body,
        grid=(GN,),
        in_specs=[
            pl.BlockSpec((BN, D), lambda i: (i, 0)),
            pl.BlockSpec((D, D), lambda i: (0, 0)),
            pl.BlockSpec((1, D), lambda i: (0, 0)),
            pl.BlockSpec((D, 2 * H), lambda i: (0, 0)),
        ],
        out_specs=[
            pl.BlockSpec((BN, WS), lambda i: (i, 0)),
            pl.BlockSpec((BN, 2 * H), lambda i: (i, 0)),
        ],
        out_shape=[
            jax.ShapeDtypeStruct((N, WS), _f32),
            jax.ShapeDtypeStruct((N, 2 * H), _f32),
        ],
    )(h, wt, b, a2)


def _rowmax_body(aa_ref, dt_ref):
    aa = aa_ref[...]
    mj = jnp.max(aa[:, H:2 * H], axis=0, keepdims=True)
    mt = aa[:, 0:H] + mj
    mt = jnp.maximum(mt, 0.2 * mt)
    dt_ref[...] = jnp.concatenate(
        [aa[:, 0:H], mt, jnp.zeros((N, WD - 2 * H), _f32)], axis=1)


def _tc_rowmax(aa):
    return pl.pallas_call(
        _rowmax_body,
        out_shape=jax.ShapeDtypeStruct((N, WD), _f32),
    )(aa)


def _combine_body(p0_ref, p1_ref, d0_ref, d1_ref, bias_ref, g_ref, bb_ref,
                  r_ref, o_ref):
    msg = p0_ref[...] + p1_ref[...]
    den = d0_ref[...] + d1_ref[...]
    inv = 1.0 / (den + 1e-16)
    drep = jnp.dot(inv, r_ref[...], preferred_element_type=_f32)
    y = msg * drep + bias_ref[...]
    y = jnp.where(y > 0.0, y, jnp.exp(jnp.minimum(y, 0.0)) - 1.0)
    mu = jnp.mean(y, axis=-1, keepdims=True)
    var = jnp.mean((y - mu) ** 2, axis=-1, keepdims=True)
    o_ref[...] = (y - mu) / jnp.sqrt(var + 1e-5) * g_ref[...] + bb_ref[...]


def _tc_combine(p0, p1, d0, d1, bias, g, bb, r):
    return pl.pallas_call(
        _combine_body,
        grid=(GN,),
        in_specs=[
            pl.BlockSpec((BN, WM), lambda i: (i, 0)),
            pl.BlockSpec((BN, WM), lambda i: (i, 0)),
            pl.BlockSpec((BN, H), lambda i: (i, 0)),
            pl.BlockSpec((BN, H), lambda i: (i, 0)),
            pl.BlockSpec((1, D), lambda i: (0, 0)),
            pl.BlockSpec((1, D), lambda i: (0, 0)),
            pl.BlockSpec((1, D), lambda i: (0, 0)),
            pl.BlockSpec((H, D), lambda i: (0, 0)),
        ],
        out_specs=pl.BlockSpec((BN, D), lambda i: (i, 0)),
        out_shape=jax.ShapeDtypeStruct((N, D), _f32),
    )(p0, p1, d0, d1, bias, g, bb, r)


@functools.cache
def _sc_mesh():
    return plsc.VectorSubcoreMesh(core_axis_name="c", subcore_axis_name="s",
                                  num_cores=NC, num_subcores=NS)


def _sc_edge_body(st_hbm, dt_hbm, sidx_hbm, didx_hbm, outm_hbm, outd_hbm,
                  sidx_v, didx_v, dring, d16ring, sbuf, dbuf, mbufv, denbuf,
                  accm, accd, *sems):
    c = lax.axis_index("c")
    s = lax.axis_index("s")
    w = s * NC + c
    sem_sg = sems[0:2]
    sem_dg = sems[2:4]
    sem_sm = sems[4:6]
    sem_sd = sems[6:8]

    pltpu.sync_copy(sidx_hbm.at[w], sidx_v)
    pltpu.sync_copy(didx_hbm.at[w], didx_v)

    def zrow(rr, carry):
        for k in range(WM // 16):
            dbuf[0, rr, pl.ds(k * 16, 16)] = jnp.zeros((16,), _f32)
        return carry

    lax.fori_loop(0, GB, zrow, 0)

    for i in range(TCHUNK):
        k = s + NS * i

        @pl.when(k < N // GB)
        def _():
            off = pl.multiple_of(k * GB, 8)
            pltpu.sync_copy(dbuf.at[0], accm.at[pl.ds(off, GB)])

        @pl.when(jnp.logical_and(k >= N // GB, k < (N + ND) // GB))
        def _():
            off = pl.multiple_of((k - N // GB) * GB, 8)
            pltpu.sync_copy(dbuf.at[0], accd.at[pl.ds(off, GB)])

    plsc.subcore_barrier()

    def gissue(j, buf):
        sv = sidx_v.at[pl.ds(j * 16, 16)]
        dv = didx_v.at[pl.ds(j * 16, 16)]
        pltpu.async_copy(st_hbm.at[sv], sbuf.at[buf], sem_sg[buf])
        pltpu.async_copy(dt_hbm.at[dv], dbuf.at[buf], sem_dg[buf])

    def gwait(j, buf):
        sv = sidx_v.at[pl.ds(j * 16, 16)]
        dv = didx_v.at[pl.ds(j * 16, 16)]
        pltpu.make_async_copy(st_hbm.at[sv], sbuf.at[buf],
                              sem_sg[buf]).wait()
        pltpu.make_async_copy(dt_hbm.at[dv], dbuf.at[buf],
                              sem_dg[buf]).wait()

    gissue(0, 0)
    gissue(1, 1)

    def swait(t):
        pltpu.make_async_copy(outm_hbm.at[c, pl.ds(0, GB)], mbufv.at[t],
                              sem_sm[t]).wait()
        pltpu.make_async_copy(outm_hbm.at[c, pl.ds(0, GB)], denbuf.at[t],
                              sem_sd[t]).wait()

    def one(j, t):
        gwait(j, t)

        @pl.when(j >= 2)
        def _():
            swait(t)

        dv = didx_v[pl.ds(j * 16, 16)]
        dring[t, :] = dv
        d16ring[t, :] = dv >> 4
        iota = lax.iota(jnp.int32, 16)
        idx_m = (iota & 7) + 8
        idx_s = iota & 7
        lof = jnp.minimum(jnp.maximum(8 - iota, 0), 1).astype(_f32)
        hif = 1.0 - lof

        def edge(k, carry):
            dvec = dbuf[t, k, 0:16]
            ajv = sbuf[t, k, pl.ds(D, 16)]
            s16 = dvec + ajv
            alpha = jnp.maximum(s16, 0.2 * s16)
            mperm = jnp.take_along_axis(s16, idx_m, axis=0)
            ex16 = jnp.exp(jnp.minimum(alpha - mperm, 0.0))
            nv = jnp.take_along_axis(dv, jnp.full((16,), k, jnp.int32),
                                     axis=0)
            slv = nv & 15
            exlo = ex16 * lof
            exhi = jnp.take_along_axis(ex16, idx_s, axis=0) * hif
            pv = (slv & 1).astype(_f32)
            val = exhi * pv + exlo * (1.0 - pv)
            qv = slv >> 1
            for q in range(8):
                qd = qv - q
                mq = (1 - jnp.minimum(qd * qd, 1)).astype(_f32)
                denbuf[t, k, pl.ds(16 * q, 16)] = val * mq
            for hh in range(8):
                g = jnp.take_along_axis(
                    ex16, jnp.full((16,), hh, jnp.int32), axis=0)
                seg = sbuf[t, k, pl.ds(16 * hh, 16)]
                mbufv[t, k, pl.ds(16 * hh, 16)] = seg * g
            return carry

        lax.fori_loop(0, 16, edge, 0)

        pltpu.async_copy(mbufv.at[t], accm.at[dring.at[t]], sem_sm[t],
                         add=True)
        pltpu.async_copy(denbuf.at[t], accd.at[d16ring.at[t]], sem_sd[t],
                         add=True)

        @pl.when(j + 2 < NBK)
        def _():
            gissue(j + 2, t)

    def body(jj, carry):
        one(jj * 2, 0)
        one(jj * 2 + 1, 1)
        return carry

    lax.fori_loop(0, NBK // 2, body, 0)
    for jt in range(NBK - 2 * (NBK // 2)):
        one(2 * (NBK // 2) + jt, jt)
    swait(0)
    swait(1)
    plsc.subcore_barrier()

    for i in range(TCHUNK):
        k = s + NS * i

        @pl.when(k < N // GB)
        def _():
            off = pl.multiple_of(k * GB, 8)
            pltpu.sync_copy(accm.at[pl.ds(off, GB)], dbuf.at[0])
            pltpu.sync_copy(dbuf.at[0], outm_hbm.at[c, pl.ds(off, GB)])

        @pl.when(jnp.logical_and(k >= N // GB, k < (N + ND) // GB))
        def _():
            off = pl.multiple_of((k - N // GB) * GB, 8)
            pltpu.sync_copy(accd.at[pl.ds(off, GB)], dbuf.at[0])
            pltpu.sync_copy(dbuf.at[0], outd_hbm.at[c, pl.ds(off, GB)])


@functools.cache
def _sc_edge():
    return pl.kernel(
        _sc_edge_body,
        out_type=(
            jax.ShapeDtypeStruct((NC, N, WM), _f32),
            jax.ShapeDtypeStruct((NC, ND, WM), _f32),
        ),
        mesh=_sc_mesh(),
        scratch_types=(
            pltpu.VMEM((EW,), jnp.int32),
            pltpu.VMEM((EW,), jnp.int32),
            pltpu.VMEM((2, 16), jnp.int32),
            pltpu.VMEM((2, 16), jnp.int32),
            pltpu.VMEM((2, 16, WS), _f32),
            pltpu.VMEM((2, 16, WM), _f32),
            pltpu.VMEM((2, 16, WM), _f32),
            pltpu.VMEM((2, 16, WM), _f32),
            pltpu.VMEM_SHARED((N, WM), _f32),
            pltpu.VMEM_SHARED((ND, WM), _f32),
        ) + (pltpu.SemaphoreType.DMA,) * 8,
    )


def kernel(x, edge_index, embed_w, embed_b, lin_w, lin_b, att, layer_bias,
           ln_g, ln_b):
    src16 = edge_index[0].reshape(NW, EW)
    dst16g = edge_index[1].reshape(NW, EW)

    eye = jnp.eye(H, dtype=_f32)
    a_dst = (att[:, :, :C, None] * eye[None, :, None, :]).reshape(L, D, H)
    a_src = (att[:, :, C:, None] * eye[None, :, None, :]).reshape(L, D, H)
    a2 = jnp.concatenate([a_dst, a_src], axis=-1)

    r = jnp.repeat(eye, C, axis=1)

    h = _tc_embed(x, embed_w.T, embed_b.reshape(1, D))
    for l in range(L):
        st, aa = _tc_transform(h, lin_w[l].T, lin_b[l].reshape(1, D), a2[l])
        dt = _tc_rowmax(aa)
        pm, pd = _sc_edge()(st, dt, src16, dst16g)
        den = pd[:, :N // 16, :].reshape(NC, N, H)
        h = _tc_combine(pm[0], pm[1], den[0], den[1],
                        layer_bias[l].reshape(1, D),
                        ln_g[l].reshape(1, D), ln_b[l].reshape(1, D), r)
    return h


# speedup vs baseline: 96.7486x; 1.7453x over previous
import functools

import jax
import jax.numpy as jnp
from jax import lax
from jax.experimental import pallas as pl
from jax.experimental.pallas import tpu as pltpu
from jax.experimental.pallas import tpu_sc as plsc

N = 10000
E = 320000
D = 128
H = 8
C = 16
L = 5

NC = 2
NS = 16
NW = NC * NS
EW = E // NW
NBK = EW // 16
GB = 16
WS = 256
WD = 128
WM = 128
ND = 640
TCHUNK = (N + ND) // GB // NS + 1

BN = 400
GN = N // BN

_f32 = jnp.float32


def _embed_body(x_ref, wt_ref, b_ref, o_ref):
    h = jnp.dot(x_ref[...], wt_ref[...], preferred_element_type=_f32)
    o_ref[...] = jnp.maximum(h + b_ref[...], 0.0)


def _tc_embed(x, wt, b):
    return pl.pallas_call(
        _embed_body,
        grid=(GN,),
        in_specs=[
            pl.BlockSpec((BN, D), lambda i: (i, 0)),
            pl.BlockSpec((D, D), lambda i: (0, 0)),
            pl.BlockSpec((1, D), lambda i: (0, 0)),
        ],
        out_specs=pl.BlockSpec((BN, D), lambda i: (i, 0)),
        out_shape=jax.ShapeDtypeStruct((N, D), _f32),
    )(x, wt, b)


def _transform_body(h_ref, wt_ref, b_ref, a2_ref, st_ref, aa_ref):
    hs = jnp.dot(h_ref[...], wt_ref[...], preferred_element_type=_f32)
    hs = hs + b_ref[...]
    aa = jnp.dot(hs, a2_ref[...], preferred_element_type=_f32)
    st_ref[...] = jnp.concatenate(
        [hs, aa[:, H:2 * H], jnp.zeros((BN, WS - D - H), _f32)], axis=1)
    aa_ref[...] = aa


def _tc_transform(h, wt, b, a2):
    return pl.pallas_call(
        _transform_body,
        grid=(GN,),
        in_specs=[
            pl.BlockSpec((BN, D), lambda i: (i, 0)),
            pl.BlockSpec((D, D), lambda i: (0, 0)),
            pl.BlockSpec((1, D), lambda i: (0, 0)),
            pl.BlockSpec((D, 2 * H), lambda i: (0, 0)),
        ],
        out_specs=[
            pl.BlockSpec((BN, WS), lambda i: (i, 0)),
            pl.BlockSpec((BN, 2 * H), lambda i: (i, 0)),
        ],
        out_shape=[
            jax.ShapeDtypeStruct((N, WS), _f32),
            jax.ShapeDtypeStruct((N, 2 * H), _f32),
        ],
    )(h, wt, b, a2)


def _rowmax_body(aa_ref, dt_ref):
    aa = aa_ref[...]
    mj = jnp.max(aa[:, H:2 * H], axis=0, keepdims=True)
    mt = aa[:, 0:H] + mj
    mt = jnp.maximum(mt, 0.2 * mt)
    dt_ref[...] = jnp.concatenate(
        [aa[:, 0:H], mt, jnp.zeros((N, WD - 2 * H), _f32)], axis=1)


def _tc_rowmax(aa):
    return pl.pallas_call(
        _rowmax_body,
        out_shape=jax.ShapeDtypeStruct((N, WD), _f32),
    )(aa)


def _combine_body(p0_ref, p1_ref, d0_ref, d1_ref, bias_ref, g_ref, bb_ref,
                  r_ref, o_ref):
    msg = p0_ref[...] + p1_ref[...]
    den = d0_ref[...] + d1_ref[...]
    inv = 1.0 / (den + 1e-16)
    drep = jnp.dot(inv, r_ref[...], preferred_element_type=_f32)
    y = msg * drep + bias_ref[...]
    y = jnp.where(y > 0.0, y, jnp.exp(jnp.minimum(y, 0.0)) - 1.0)
    mu = jnp.mean(y, axis=-1, keepdims=True)
    var = jnp.mean((y - mu) ** 2, axis=-1, keepdims=True)
    o_ref[...] = (y - mu) / jnp.sqrt(var + 1e-5) * g_ref[...] + bb_ref[...]


def _tc_combine(p0, p1, d0, d1, bias, g, bb, r):
    return pl.pallas_call(
        _combine_body,
        grid=(GN,),
        in_specs=[
            pl.BlockSpec((BN, WM), lambda i: (i, 0)),
            pl.BlockSpec((BN, WM), lambda i: (i, 0)),
            pl.BlockSpec((BN, H), lambda i: (i, 0)),
            pl.BlockSpec((BN, H), lambda i: (i, 0)),
            pl.BlockSpec((1, D), lambda i: (0, 0)),
            pl.BlockSpec((1, D), lambda i: (0, 0)),
            pl.BlockSpec((1, D), lambda i: (0, 0)),
            pl.BlockSpec((H, D), lambda i: (0, 0)),
        ],
        out_specs=pl.BlockSpec((BN, D), lambda i: (i, 0)),
        out_shape=jax.ShapeDtypeStruct((N, D), _f32),
    )(p0, p1, d0, d1, bias, g, bb, r)


@functools.cache
def _sc_mesh():
    return plsc.VectorSubcoreMesh(core_axis_name="c", subcore_axis_name="s",
                                  num_cores=NC, num_subcores=NS)


def _sc_edge_body(st_hbm, dt_hbm, sidx_hbm, didx_hbm, outm_hbm, outd_hbm,
                  sidx_v, didx_v, dring, d16ring, sbuf, dbuf, mbufv, denbuf,
                  accm, accd, *sems):
    c = lax.axis_index("c")
    s = lax.axis_index("s")
    w = s * NC + c
    sem_sg = sems[0:2]
    sem_dg = sems[2:4]
    sem_sm = sems[4:6]
    sem_sd = sems[6:8]

    pltpu.sync_copy(sidx_hbm.at[w], sidx_v)
    pltpu.sync_copy(didx_hbm.at[w], didx_v)

    def zrow(rr, carry):
        for k in range(WM // 16):
            dbuf[0, rr, pl.ds(k * 16, 16)] = jnp.zeros((16,), _f32)
        return carry

    lax.fori_loop(0, GB, zrow, 0)

    for i in range(TCHUNK):
        k = s + NS * i

        @pl.when(k < N // GB)
        def _():
            off = pl.multiple_of(k * GB, 8)
            pltpu.sync_copy(dbuf.at[0], accm.at[pl.ds(off, GB)])

        @pl.when(jnp.logical_and(k >= N // GB, k < (N + ND) // GB))
        def _():
            off = pl.multiple_of((k - N // GB) * GB, 8)
            pltpu.sync_copy(dbuf.at[0], accd.at[pl.ds(off, GB)])

    plsc.subcore_barrier()

    def gissue(j, buf):
        sv = sidx_v.at[pl.ds(j * 16, 16)]
        dv = didx_v.at[pl.ds(j * 16, 16)]
        pltpu.async_copy(st_hbm.at[sv], sbuf.at[buf], sem_sg[buf])
        pltpu.async_copy(dt_hbm.at[dv], dbuf.at[buf], sem_dg[buf])

    def gwait(j, buf):
        sv = sidx_v.at[pl.ds(j * 16, 16)]
        dv = didx_v.at[pl.ds(j * 16, 16)]
        pltpu.make_async_copy(st_hbm.at[sv], sbuf.at[buf],
                              sem_sg[buf]).wait()
        pltpu.make_async_copy(dt_hbm.at[dv], dbuf.at[buf],
                              sem_dg[buf]).wait()

    gissue(0, 0)
    gissue(1, 1)

    def swait(t):
        pltpu.make_async_copy(outm_hbm.at[c, pl.ds(0, GB)], mbufv.at[t],
                              sem_sm[t]).wait()
        pltpu.make_async_copy(outm_hbm.at[c, pl.ds(0, GB)], denbuf.at[t],
                              sem_sd[t]).wait()

    def one(j, t):
        gwait(j, t)

        @pl.when(j >= 2)
        def _():
            swait(t)

        dv = didx_v[pl.ds(j * 16, 16)]
        dring[t, :] = dv
        d16ring[t, :] = dv >> 4
        iota = lax.iota(jnp.int32, 16)
        idx_m = (iota & 7) + 8
        idx_s = iota & 7
        lof = jnp.minimum(jnp.maximum(8 - iota, 0), 1).astype(_f32)
        hif = 1.0 - lof

        @functools.partial(plsc.parallel_loop, 0, 16, unroll=4)
        def edge(k):
            dvec = dbuf[t, k, 0:16]
            ajv = sbuf[t, k, pl.ds(D, 16)]
            s16 = dvec + ajv
            alpha = jnp.maximum(s16, 0.2 * s16)
            mperm = jnp.take_along_axis(s16, idx_m, axis=0)
            ex16 = jnp.exp(jnp.minimum(alpha - mperm, 0.0))
            nv = jnp.take_along_axis(dv, jnp.full((16,), k, jnp.int32),
                                     axis=0)
            slv = nv & 15
            exlo = ex16 * lof
            exhi = jnp.take_along_axis(ex16, idx_s, axis=0) * hif
            pv = (slv & 1).astype(_f32)
            val = exhi * pv + exlo * (1.0 - pv)
            qv = slv >> 1
            for q in range(8):
                qd = qv - q
                mq = (1 - jnp.minimum(qd * qd, 1)).astype(_f32)
                denbuf[t, k, pl.ds(16 * q, 16)] = val * mq
            for hh in range(8):
                g = jnp.take_along_axis(
                    ex16, jnp.full((16,), hh, jnp.int32), axis=0)
                seg = sbuf[t, k, pl.ds(16 * hh, 16)]
                mbufv[t, k, pl.ds(16 * hh, 16)] = seg * g

        pltpu.async_copy(mbufv.at[t], accm.at[dring.at[t]], sem_sm[t],
                         add=True)
        pltpu.async_copy(denbuf.at[t], accd.at[d16ring.at[t]], sem_sd[t],
                         add=True)

        @pl.when(j + 2 < NBK)
        def _():
            gissue(j + 2, t)

    def body(jj, carry):
        one(jj * 2, 0)
        one(jj * 2 + 1, 1)
        return carry

    lax.fori_loop(0, NBK // 2, body, 0)
    for jt in range(NBK - 2 * (NBK // 2)):
        one(2 * (NBK // 2) + jt, jt)
    swait(0)
    swait(1)
    plsc.subcore_barrier()

    for i in range(TCHUNK):
        k = s + NS * i

        @pl.when(k < N // GB)
        def _():
            off = pl.multiple_of(k * GB, 8)
            pltpu.sync_copy(accm.at[pl.ds(off, GB)], dbuf.at[0])
            pltpu.sync_copy(dbuf.at[0], outm_hbm.at[c, pl.ds(off, GB)])

        @pl.when(jnp.logical_and(k >= N // GB, k < (N + ND) // GB))
        def _():
            off = pl.multiple_of((k - N // GB) * GB, 8)
            pltpu.sync_copy(accd.at[pl.ds(off, GB)], dbuf.at[0])
            pltpu.sync_copy(dbuf.at[0], outd_hbm.at[c, pl.ds(off, GB)])


@functools.cache
def _sc_edge():
    return pl.kernel(
        _sc_edge_body,
        out_type=(
            jax.ShapeDtypeStruct((NC, N, WM), _f32),
            jax.ShapeDtypeStruct((NC, ND, WM), _f32),
        ),
        mesh=_sc_mesh(),
        scratch_types=(
            pltpu.VMEM((EW,), jnp.int32),
            pltpu.VMEM((EW,), jnp.int32),
            pltpu.VMEM((2, 16), jnp.int32),
            pltpu.VMEM((2, 16), jnp.int32),
            pltpu.VMEM((2, 16, WS), _f32),
            pltpu.VMEM((2, 16, WM), _f32),
            pltpu.VMEM((2, 16, WM), _f32),
            pltpu.VMEM((2, 16, WM), _f32),
            pltpu.VMEM_SHARED((N, WM), _f32),
            pltpu.VMEM_SHARED((ND, WM), _f32),
        ) + (pltpu.SemaphoreType.DMA,) * 8,
    )


def kernel(x, edge_index, embed_w, embed_b, lin_w, lin_b, att, layer_bias,
           ln_g, ln_b):
    src16 = edge_index[0].reshape(NW, EW)
    dst16g = edge_index[1].reshape(NW, EW)

    eye = jnp.eye(H, dtype=_f32)
    a_dst = (att[:, :, :C, None] * eye[None, :, None, :]).reshape(L, D, H)
    a_src = (att[:, :, C:, None] * eye[None, :, None, :]).reshape(L, D, H)
    a2 = jnp.concatenate([a_dst, a_src], axis=-1)

    r = jnp.repeat(eye, C, axis=1)

    h = _tc_embed(x, embed_w.T, embed_b.reshape(1, D))
    for l in range(L):
        st, aa = _tc_transform(h, lin_w[l].T, lin_b[l].reshape(1, D), a2[l])
        dt = _tc_rowmax(aa)
        pm, pd = _sc_edge()(st, dt, src16, dst16g)
        den = pd[:, :N // 16, :].reshape(NC, N, H)
        h = _tc_combine(pm[0], pm[1], den[0], den[1],
                        layer_bias[l].reshape(1, D),
                        ln_g[l].reshape(1, D), ln_b[l].reshape(1, D), r)
    return h
